# baseline (speedup 1.0000x reference)
import functools
import math

import jax
import jax.numpy as jnp
from jax import lax
from jax.experimental import pallas as pl
from jax.experimental.pallas import tpu as pltpu
from jax.experimental.pallas import tpu_sc as plsc

_VOCAB = 1000000
_D = 64
_SEQ = 50
_BATCH = 4096
_NC = 2
_NS = 16
_NW = _NC * _NS
_N = _BATCH * _SEQ
_C = 128
_PER_W = _N // _NW
_NCHUNK = _PER_W // _C
_SCALE = math.sqrt(_D)
_LANES = 16
_DV = _D // _LANES


def _body(idx_hbm, tab_hbm, pe_hbm, out_hbm, idx_v, pe_v, buf0, buf1,
          gsem0, gsem1, ssem0, ssem1):
    cid = lax.axis_index("c")
    sid = lax.axis_index("s")
    wid = sid * _NC + cid
    pltpu.sync_copy(idx_hbm.at[wid], idx_v)
    pltpu.sync_copy(pe_hbm, pe_v)
    row_base = wid * _PER_W
    bufs = (buf0, buf1)
    gsems = (gsem0, gsem1)
    ssems = (ssem0, ssem1)

    pltpu.async_copy(tab_hbm.at[idx_v.at[0]], buf0, gsem0)

    @pl.loop(0, _NCHUNK, step=2)
    def _pair(j0):
        for par in range(2):
            j = j0 + par
            buf, gsem, ssem = bufs[par], gsems[par], ssems[par]
            nbuf, ngsem = bufs[1 - par], gsems[1 - par]
            nssem = ssems[1 - par]

            @pl.when(j + 1 < _NCHUNK)
            def _start_next():
                @pl.when(j >= 1)
                def _drain_prev():
                    pltpu.make_async_copy(
                        nbuf, out_hbm.at[pl.ds(row_base + (j - 1) * _C, _C)],
                        nssem).wait()

                pltpu.async_copy(tab_hbm.at[idx_v.at[j + 1]], nbuf, ngsem)

            pltpu.make_async_copy(tab_hbm.at[idx_v.at[j]], buf, gsem).wait()

            p0 = lax.rem(j * _C, _SEQ)

            @pl.loop(0, _C, unroll=4)
            def _row(r):
                p = p0 + r
                for t in range(_DV):
                    sl = pl.ds(t * _LANES, _LANES)
                    buf[r, sl] = buf[r, sl] * _SCALE + pe_v[p, sl]

            pltpu.async_copy(
                buf, out_hbm.at[pl.ds(row_base + j * _C, _C)], ssem)

    pltpu.make_async_copy(
        buf0, out_hbm.at[pl.ds(row_base + (_NCHUNK - 2) * _C, _C)],
        ssem0).wait()
    pltpu.make_async_copy(
        buf1, out_hbm.at[pl.ds(row_base + (_NCHUNK - 1) * _C, _C)],
        ssem1).wait()


@functools.partial(jax.jit, static_argnames=())
def kernel(enc_words, table, pe):
    idx = enc_words.reshape(_NW, _NCHUNK, _C).astype(jnp.int32)
    pe2 = pe.reshape(_SEQ, _D).astype(jnp.float32)
    pe2 = jnp.tile(pe2, (4, 1))
    mesh = plsc.VectorSubcoreMesh(core_axis_name="c", subcore_axis_name="s")
    out = pl.kernel(
        _body,
        out_type=jax.ShapeDtypeStruct((_N, _D), jnp.float32),
        mesh=mesh,
        compiler_params=pltpu.CompilerParams(use_tc_tiling_on_sc=False),
        scratch_types=[
            pltpu.VMEM((_NCHUNK, _C), jnp.int32),
            pltpu.VMEM((4 * _SEQ, _D), jnp.float32),
            pltpu.VMEM((_C, _D), jnp.float32),
            pltpu.VMEM((_C, _D), jnp.float32),
            pltpu.SemaphoreType.DMA,
            pltpu.SemaphoreType.DMA,
            pltpu.SemaphoreType.DMA,
            pltpu.SemaphoreType.DMA,
        ],
    )(idx, table, pe2)
    return out.reshape(_BATCH, _SEQ, _D)
